# Initial kernel scaffold; baseline (speedup 1.0000x reference)
import jax
import jax.numpy as jnp
from jax.experimental import pallas as pl


def kernel(features, initial_features, edge_index):
    raise NotImplementedError("write your pallas kernel here")


# speedup vs baseline: 2.7325x; 2.7325x over previous
import functools

import jax
import jax.numpy as jnp
from jax import lax
from jax.experimental import pallas as pl
from jax.experimental.pallas import tpu as pltpu
from jax.experimental.pallas import tpu_sc as plsc

N = 10000
D = 128
E = 320000
K = 2
ALPHA = 0.1

NC = 2
NS = 16
CH = 128
NROWS = E // CH + (-(E // CH) % (NC * NS * 8))
EPAD = NROWS * CH
RPT = NROWS // (NC * NS)
NPAD = N + (-N % (NS * 8))
RZ = NPAD // NS

_mesh = plsc.VectorSubcoreMesh(core_axis_name="c", subcore_axis_name="s")


@functools.partial(
    pl.kernel,
    out_type=jax.ShapeDtypeStruct((NC, NPAD, D), jnp.float32),
    mesh=_mesh,
    scratch_types=[
        pltpu.VMEM_SHARED((NPAD, D), jnp.float32),
        pltpu.VMEM((RPT, CH), jnp.int32),
        pltpu.VMEM((CH, D), jnp.float32),
    ],
)
def _deg(dst_hbm, out_hbm, dtab, dst_v, buf):
    c = lax.axis_index("c")
    s = lax.axis_index("s")
    wid = c * NS + s
    pltpu.sync_copy(dst_hbm.at[pl.ds(wid * RPT, RPT)], dst_v)

    zero = jnp.zeros((16,), jnp.float32)

    def _zrow(r, carry):
        for k in range(D // 16):
            buf[r, pl.ds(k * 16, 16)] = zero
        return carry

    lax.fori_loop(0, CH, _zrow, 0)
    r0 = s * RZ
    nfull = RZ // CH
    for t in range(nfull):
        pltpu.sync_copy(buf, dtab.at[pl.ds(r0 + t * CH, CH)])
    rem = RZ - nfull * CH
    if rem:
        pltpu.sync_copy(buf.at[pl.ds(0, rem)], dtab.at[pl.ds(r0 + nfull * CH, rem)])

    one = jnp.ones((16,), jnp.float32)

    def _orow(r, carry):
        for k in range(D // 16):
            buf[r, pl.ds(k * 16, 16)] = one
        return carry

    lax.fori_loop(0, CH, _orow, 0)
    plsc.subcore_barrier()

    def _chunk(j, carry):
        pltpu.sync_copy(buf, dtab.at[dst_v.at[j]], add=True)
        return carry

    lax.fori_loop(0, RPT, _chunk, 0)
    plsc.subcore_barrier()
    pltpu.sync_copy(dtab.at[pl.ds(r0, RZ)], out_hbm.at[c, pl.ds(r0, RZ)])


@functools.partial(
    pl.kernel,
    out_type=jax.ShapeDtypeStruct((NC, NPAD, D), jnp.float32),
    mesh=_mesh,
    scratch_types=[
        pltpu.VMEM_SHARED((NPAD, D), jnp.float32),
        pltpu.VMEM((RPT, CH), jnp.int32),
        pltpu.VMEM((RPT, CH), jnp.int32),
        pltpu.VMEM((CH, D), jnp.float32),
        pltpu.SemaphoreType.DMA,
    ],
)
def _prop(g_hbm, src_hbm, dst_hbm, out_hbm, acc, src_v, dst_v, rowbuf, sem):
    c = lax.axis_index("c")
    s = lax.axis_index("s")
    wid = c * NS + s
    pltpu.sync_copy(src_hbm.at[pl.ds(wid * RPT, RPT)], src_v)
    pltpu.sync_copy(dst_hbm.at[pl.ds(wid * RPT, RPT)], dst_v)

    zero = jnp.zeros((16,), jnp.float32)

    def _zrow(r, carry):
        for k in range(D // 16):
            rowbuf[r, pl.ds(k * 16, 16)] = zero
        return carry

    lax.fori_loop(0, CH, _zrow, 0)
    r0 = s * RZ
    nfull = RZ // CH
    for t in range(nfull):
        pltpu.sync_copy(rowbuf, acc.at[pl.ds(r0 + t * CH, CH)])
    rem = RZ - nfull * CH
    if rem:
        pltpu.sync_copy(rowbuf.at[pl.ds(0, rem)], acc.at[pl.ds(r0 + nfull * CH, rem)])
    plsc.subcore_barrier()

    def _chunk(j, carry):
        pltpu.async_copy(g_hbm.at[src_v.at[j]], rowbuf, sem).wait()
        pltpu.sync_copy(rowbuf, acc.at[dst_v.at[j]], add=True)
        return carry

    lax.fori_loop(0, RPT, _chunk, 0)
    plsc.subcore_barrier()
    pltpu.sync_copy(acc.at[pl.ds(r0, RZ)], out_hbm.at[c, pl.ds(r0, RZ)])


BLK = 1000


def _prep_body(pdeg_ref, init_ref, norm_ref, g_ref, r_ref):
    degs = pdeg_ref[0, :, 0:1] + pdeg_ref[1, :, 0:1]
    norm = lax.rsqrt(degs + 1.0)
    norm_ref[...] = norm
    x = init_ref[...]
    g_ref[...] = x * norm
    r_ref[...] = ALPHA * x * (norm * norm)


def _prep(pdeg, init):
    return pl.pallas_call(
        _prep_body,
        grid=(N // BLK,),
        in_specs=[
            pl.BlockSpec((NC, BLK, D), lambda i: (0, i, 0)),
            pl.BlockSpec((BLK, D), lambda i: (i, 0)),
        ],
        out_specs=[
            pl.BlockSpec((BLK, 1), lambda i: (i, 0)),
            pl.BlockSpec((BLK, D), lambda i: (i, 0)),
            pl.BlockSpec((BLK, D), lambda i: (i, 0)),
        ],
        out_shape=[
            jax.ShapeDtypeStruct((N, 1), jnp.float32),
            jax.ShapeDtypeStruct((N, D), jnp.float32),
            jax.ShapeDtypeStruct((N, D), jnp.float32),
        ],
    )(pdeg, init)


def _blend_body(emit_g, p_ref, hpre_ref, r_ref, norm_ref, *out_refs):
    norm = norm_ref[...]
    h = ALPHA * ((p_ref[0] + p_ref[1]) * norm) + r_ref[...] + (1.0 - ALPHA) * hpre_ref[...]
    out_refs[0][...] = h
    if emit_g:
        out_refs[1][...] = h * norm


def _blend(p, hpre, r, norm, emit_g):
    n_out = 2 if emit_g else 1
    return pl.pallas_call(
        functools.partial(_blend_body, emit_g),
        grid=(N // BLK,),
        in_specs=[
            pl.BlockSpec((NC, BLK, D), lambda i: (0, i, 0)),
            pl.BlockSpec((BLK, D), lambda i: (i, 0)),
            pl.BlockSpec((BLK, D), lambda i: (i, 0)),
            pl.BlockSpec((BLK, 1), lambda i: (i, 0)),
        ],
        out_specs=[pl.BlockSpec((BLK, D), lambda i: (i, 0))] * n_out,
        out_shape=[jax.ShapeDtypeStruct((N, D), jnp.float32)] * n_out,
    )(p, hpre, r, norm)


def kernel(features, initial_features, edge_index):
    del features
    ei = edge_index.astype(jnp.int32)
    npad_e = EPAD - E
    src = jnp.concatenate([ei[0], jnp.zeros((npad_e,), jnp.int32)]).reshape(NROWS, CH)
    dst = jnp.concatenate([ei[1], jnp.full((npad_e,), N, jnp.int32)]).reshape(NROWS, CH)

    pdeg = _deg(dst)
    norm, g, r = _prep(pdeg, initial_features)

    h_pre = initial_features
    for step in range(K):
        p = _prop(g, src, dst)
        out = _blend(p, h_pre, r, norm, emit_g=(step < K - 1))
        if step < K - 1:
            h_pre, g = out
        else:
            (h,) = out
    return h


# speedup vs baseline: 3.0021x; 1.0987x over previous
import functools

import jax
import jax.numpy as jnp
from jax import lax
from jax.experimental import pallas as pl
from jax.experimental.pallas import tpu as pltpu
from jax.experimental.pallas import tpu_sc as plsc

N = 10000
D = 128
E = 320000
K = 2
ALPHA = 0.1

NC = 2
NS = 16
CH = 64
NROWS = E // CH + (-(E // CH) % (NC * NS * 8))
EPAD = NROWS * CH
RPT = NROWS // (NC * NS)
NPAD = N + (-N % (NS * 8))
RZ = NPAD // NS

_mesh = plsc.VectorSubcoreMesh(core_axis_name="c", subcore_axis_name="s")


@functools.partial(
    pl.kernel,
    out_type=jax.ShapeDtypeStruct((NC, NPAD, D), jnp.float32),
    mesh=_mesh,
    scratch_types=[
        pltpu.VMEM_SHARED((NPAD, D), jnp.float32),
        pltpu.VMEM((RPT, CH), jnp.int32),
        pltpu.VMEM((CH, D), jnp.float32),
        pltpu.SemaphoreType.DMA,
    ],
)
def _deg(dst_hbm, out_hbm, dtab, dst_v, buf, sem):
    c = lax.axis_index("c")
    s = lax.axis_index("s")
    wid = c * NS + s
    pltpu.sync_copy(dst_hbm.at[pl.ds(wid * RPT, RPT)], dst_v)

    zero = jnp.zeros((16,), jnp.float32)

    def _zrow(r, carry):
        for k in range(D // 16):
            buf[r, pl.ds(k * 16, 16)] = zero
        return carry

    lax.fori_loop(0, CH, _zrow, 0)
    r0 = s * RZ
    nfull = RZ // CH
    for t in range(nfull):
        pltpu.sync_copy(buf, dtab.at[pl.ds(r0 + t * CH, CH)])
    rem = RZ - nfull * CH
    if rem:
        pltpu.sync_copy(buf.at[pl.ds(0, rem)], dtab.at[pl.ds(r0 + nfull * CH, rem)])

    one = jnp.ones((16,), jnp.float32)

    def _orow(r, carry):
        for k in range(D // 16):
            buf[r, pl.ds(k * 16, 16)] = one
        return carry

    lax.fori_loop(0, CH, _orow, 0)
    plsc.subcore_barrier()

    W = 8

    def _wave(w, carry):
        descs = [
            pltpu.async_copy(buf, dtab.at[dst_v.at[w * W + b]], sem, add=True)
            for b in range(W)
        ]
        for d in descs:
            d.wait()
        return carry

    lax.fori_loop(0, RPT // W, _wave, 0)
    plsc.subcore_barrier()
    pltpu.sync_copy(dtab.at[pl.ds(r0, RZ)], out_hbm.at[c, pl.ds(r0, RZ)])


NBUF = 5
assert RPT % NBUF == 0
NGRP = RPT // NBUF


@functools.partial(
    pl.kernel,
    out_type=jax.ShapeDtypeStruct((NC, NPAD, D), jnp.float32),
    mesh=_mesh,
    scratch_types=[
        pltpu.VMEM_SHARED((NPAD, D), jnp.float32),
        pltpu.VMEM((NBUF, CH), jnp.int32),
        pltpu.VMEM((NBUF, CH), jnp.int32),
        pltpu.VMEM((NBUF, CH, D), jnp.float32),
        [pltpu.SemaphoreType.DMA] * NBUF,
        [pltpu.SemaphoreType.DMA] * NBUF,
        [pltpu.SemaphoreType.DMA] * NBUF,
    ],
)
def _prop(g_hbm, src_hbm, dst_hbm, out_hbm, acc, idx_s, idx_d, bufs, isems, gsems, ssems):
    c = lax.axis_index("c")
    s = lax.axis_index("s")
    wid = c * NS + s
    base = wid * RPT

    zero = jnp.zeros((16,), jnp.float32)
    zbuf = bufs.at[0]

    def _zrow(r, carry):
        for k in range(D // 16):
            zbuf[r, pl.ds(k * 16, 16)] = zero
        return carry

    lax.fori_loop(0, CH, _zrow, 0)
    r0 = s * RZ
    nfull = RZ // CH
    for t in range(nfull):
        pltpu.sync_copy(zbuf, acc.at[pl.ds(r0 + t * CH, CH)])
    rem = RZ - nfull * CH
    if rem:
        pltpu.sync_copy(zbuf.at[pl.ds(0, rem)], acc.at[pl.ds(r0 + nfull * CH, rem)])
    plsc.subcore_barrier()

    def _start_idx(j, b):
        pltpu.async_copy(src_hbm.at[base + j], idx_s.at[b], isems[b])
        pltpu.async_copy(dst_hbm.at[base + j], idx_d.at[b], isems[b])

    def _wait_idx(j, b):
        pltpu.make_async_copy(src_hbm.at[base + j], idx_s.at[b], isems[b]).wait()
        pltpu.make_async_copy(dst_hbm.at[base + j], idx_d.at[b], isems[b]).wait()

    def _start_gather(b):
        pltpu.async_copy(g_hbm.at[idx_s.at[b]], bufs.at[b], gsems[b])

    def _wait_gather(b):
        pltpu.make_async_copy(g_hbm.at[idx_s.at[b]], bufs.at[b], gsems[b]).wait()

    for b in range(NBUF):
        _start_idx(b, b)
    for b in range(NBUF):
        _wait_idx(b, b)
        _start_gather(b)

    def _grp(i, carry):
        sdescs = []
        for b in range(NBUF):
            j = i * NBUF + b
            _wait_gather(b)
            sdescs.append(
                pltpu.async_copy(bufs.at[b], acc.at[idx_d.at[b]], ssems[b], add=True)
            )
        for b in range(NBUF):
            sdescs[b].wait()

            @pl.when(i + 1 < NGRP)
            def _():
                _start_idx((i + 1) * NBUF + b, b)

        for b in range(NBUF):
            @pl.when(i + 1 < NGRP)
            def _():
                _wait_idx((i + 1) * NBUF + b, b)
                _start_gather(b)

        return carry

    lax.fori_loop(0, NGRP, _grp, 0)
    plsc.subcore_barrier()
    pltpu.sync_copy(acc.at[pl.ds(r0, RZ)], out_hbm.at[c, pl.ds(r0, RZ)])


BLK = 1000


def _prep_body(pdeg_ref, init_ref, norm_ref, g_ref, r_ref):
    degs = pdeg_ref[0, :, 0:1] + pdeg_ref[1, :, 0:1]
    norm = lax.rsqrt(degs + 1.0)
    norm_ref[...] = norm
    x = init_ref[...]
    g_ref[...] = x * norm
    r_ref[...] = ALPHA * x * (norm * norm)


def _prep(pdeg, init):
    return pl.pallas_call(
        _prep_body,
        grid=(N // BLK,),
        in_specs=[
            pl.BlockSpec((NC, BLK, D), lambda i: (0, i, 0)),
            pl.BlockSpec((BLK, D), lambda i: (i, 0)),
        ],
        out_specs=[
            pl.BlockSpec((BLK, 1), lambda i: (i, 0)),
            pl.BlockSpec((BLK, D), lambda i: (i, 0)),
            pl.BlockSpec((BLK, D), lambda i: (i, 0)),
        ],
        out_shape=[
            jax.ShapeDtypeStruct((N, 1), jnp.float32),
            jax.ShapeDtypeStruct((N, D), jnp.float32),
            jax.ShapeDtypeStruct((N, D), jnp.float32),
        ],
    )(pdeg, init)


def _blend_body(emit_g, p_ref, hpre_ref, r_ref, norm_ref, *out_refs):
    norm = norm_ref[...]
    h = ALPHA * ((p_ref[0] + p_ref[1]) * norm) + r_ref[...] + (1.0 - ALPHA) * hpre_ref[...]
    out_refs[0][...] = h
    if emit_g:
        out_refs[1][...] = h * norm


def _blend(p, hpre, r, norm, emit_g):
    n_out = 2 if emit_g else 1
    return pl.pallas_call(
        functools.partial(_blend_body, emit_g),
        grid=(N // BLK,),
        in_specs=[
            pl.BlockSpec((NC, BLK, D), lambda i: (0, i, 0)),
            pl.BlockSpec((BLK, D), lambda i: (i, 0)),
            pl.BlockSpec((BLK, D), lambda i: (i, 0)),
            pl.BlockSpec((BLK, 1), lambda i: (i, 0)),
        ],
        out_specs=[pl.BlockSpec((BLK, D), lambda i: (i, 0))] * n_out,
        out_shape=[jax.ShapeDtypeStruct((N, D), jnp.float32)] * n_out,
    )(p, hpre, r, norm)


def kernel(features, initial_features, edge_index):
    del features
    ei = edge_index.astype(jnp.int32)
    npad_e = EPAD - E
    src = jnp.concatenate([ei[0], jnp.zeros((npad_e,), jnp.int32)]).reshape(NROWS, CH)
    dst = jnp.concatenate([ei[1], jnp.full((npad_e,), N, jnp.int32)]).reshape(NROWS, CH)

    pdeg = _deg(dst)
    norm, g, r = _prep(pdeg, initial_features)

    h_pre = initial_features
    for step in range(K):
        p = _prop(g, src, dst)
        out = _blend(p, h_pre, r, norm, emit_g=(step < K - 1))
        if step < K - 1:
            h_pre, g = out
        else:
            (h,) = out
    return h


# speedup vs baseline: 8.3687x; 2.7876x over previous
import functools

import jax
import jax.numpy as jnp
from jax import lax
from jax.experimental import pallas as pl
from jax.experimental.pallas import tpu as pltpu
from jax.experimental.pallas import tpu_sc as plsc

N = 10000
D = 128
E = 320000
K = 2
ALPHA = 0.1

NC = 2
NS = 16
CH = 64
NROWS = E // CH + (-(E // CH) % (NC * NS * 8))
EPAD = NROWS * CH
RPT = NROWS // (NC * NS)
NPAD = N + (-N % (NS * 8))
RZ = NPAD // NS

_mesh = plsc.VectorSubcoreMesh(core_axis_name="c", subcore_axis_name="s")


@functools.partial(
    pl.kernel,
    out_type=jax.ShapeDtypeStruct((NC, NPAD, D), jnp.float32),
    mesh=_mesh,
    scratch_types=[
        pltpu.VMEM_SHARED((NPAD, D), jnp.float32),
        pltpu.VMEM((RPT, CH), jnp.int32),
        pltpu.VMEM((CH, D), jnp.float32),
        pltpu.SemaphoreType.DMA,
    ],
)
def _deg(dst_hbm, out_hbm, dtab, dst_v, buf, sem):
    c = lax.axis_index("c")
    s = lax.axis_index("s")
    wid = c * NS + s
    pltpu.sync_copy(dst_hbm.at[pl.ds(wid * RPT, RPT)], dst_v)

    zero = jnp.zeros((16,), jnp.float32)

    def _zrow(r, carry):
        for k in range(D // 16):
            buf[r, pl.ds(k * 16, 16)] = zero
        return carry

    lax.fori_loop(0, CH, _zrow, 0)
    r0 = s * RZ
    nfull = RZ // CH
    for t in range(nfull):
        pltpu.sync_copy(buf, dtab.at[pl.ds(r0 + t * CH, CH)])
    rem = RZ - nfull * CH
    if rem:
        pltpu.sync_copy(buf.at[pl.ds(0, rem)], dtab.at[pl.ds(r0 + nfull * CH, rem)])

    one = jnp.ones((16,), jnp.float32)

    def _orow(r, carry):
        for k in range(D // 16):
            buf[r, pl.ds(k * 16, 16)] = one
        return carry

    lax.fori_loop(0, CH, _orow, 0)
    plsc.subcore_barrier()

    W = 8

    def _wave(w, carry):
        descs = [
            pltpu.async_copy(buf, dtab.at[dst_v.at[w * W + b]], sem, add=True)
            for b in range(W)
        ]
        for d in descs:
            d.wait()
        return carry

    lax.fori_loop(0, RPT // W, _wave, 0)
    plsc.subcore_barrier()
    pltpu.sync_copy(dtab.at[pl.ds(r0, RZ)], out_hbm.at[c, pl.ds(r0, RZ)])


NBUF = 5
assert RPT % NBUF == 0
NGRP = RPT // NBUF


@functools.partial(
    pl.kernel,
    out_type=jax.ShapeDtypeStruct((NC, NPAD, D), jnp.float32),
    mesh=_mesh,
    scratch_types=[
        pltpu.VMEM_SHARED((NPAD, D), jnp.float32),
        pltpu.VMEM((NBUF, CH), jnp.int32),
        pltpu.VMEM((NBUF, CH), jnp.int32),
        pltpu.VMEM((NBUF, CH, D), jnp.float32),
        [pltpu.SemaphoreType.DMA] * NBUF,
        [pltpu.SemaphoreType.DMA] * NBUF,
        [pltpu.SemaphoreType.DMA] * NBUF,
    ],
)
def _prop(g_hbm, src_hbm, dst_hbm, out_hbm, acc, idx_s, idx_d, bufs, isems, gsems, ssems):
    c = lax.axis_index("c")
    s = lax.axis_index("s")
    wid = c * NS + s
    base = wid * RPT

    zero = jnp.zeros((16,), jnp.float32)
    zbuf = bufs.at[0]

    def _zrow(r, carry):
        for k in range(D // 16):
            zbuf[r, pl.ds(k * 16, 16)] = zero
        return carry

    lax.fori_loop(0, CH, _zrow, 0)
    r0 = s * RZ
    nfull = RZ // CH
    for t in range(nfull):
        pltpu.sync_copy(zbuf, acc.at[pl.ds(r0 + t * CH, CH)])
    rem = RZ - nfull * CH
    if rem:
        pltpu.sync_copy(zbuf.at[pl.ds(0, rem)], acc.at[pl.ds(r0 + nfull * CH, rem)])
    plsc.subcore_barrier()

    def _start_idx(j, b):
        pltpu.async_copy(src_hbm.at[base + j], idx_s.at[b], isems[b])
        pltpu.async_copy(dst_hbm.at[base + j], idx_d.at[b], isems[b])

    def _wait_idx(j, b):
        pltpu.make_async_copy(src_hbm.at[base + j], idx_s.at[b], isems[b]).wait()
        pltpu.make_async_copy(dst_hbm.at[base + j], idx_d.at[b], isems[b]).wait()

    def _start_gather(b):
        pltpu.async_copy(g_hbm.at[idx_s.at[b]], bufs.at[b], gsems[b])

    def _wait_gather(b):
        pltpu.make_async_copy(g_hbm.at[idx_s.at[b]], bufs.at[b], gsems[b]).wait()

    for b in range(NBUF):
        _start_idx(b, b)
    for b in range(NBUF):
        _wait_idx(b, b)
        _start_gather(b)

    def _grp(i, carry):
        sdescs = []
        for b in range(NBUF):
            j = i * NBUF + b
            _wait_gather(b)
            sdescs.append(
                pltpu.async_copy(bufs.at[b], acc.at[idx_d.at[b]], ssems[b], add=True)
            )
        for b in range(NBUF):
            sdescs[b].wait()

            @pl.when(i + 1 < NGRP)
            def _():
                _start_idx((i + 1) * NBUF + b, b)

        for b in range(NBUF):
            @pl.when(i + 1 < NGRP)
            def _():
                _wait_idx((i + 1) * NBUF + b, b)
                _start_gather(b)

        return carry

    lax.fori_loop(0, NGRP, _grp, 0)
    plsc.subcore_barrier()
    pltpu.sync_copy(acc.at[pl.ds(r0, RZ)], out_hbm.at[c, pl.ds(r0, RZ)])


BLK = 1000


def _prep_body(pdeg_ref, init_ref, norm_ref, g_ref, r_ref):
    degs = pdeg_ref[0, :, 0:1] + pdeg_ref[1, :, 0:1]
    norm = lax.rsqrt(degs + 1.0)
    norm_ref[...] = norm
    x = init_ref[...]
    g_ref[...] = x * norm
    r_ref[...] = ALPHA * x * (norm * norm)


def _prep(pdeg, init):
    return pl.pallas_call(
        _prep_body,
        grid=(N // BLK,),
        in_specs=[
            pl.BlockSpec((NC, BLK, D), lambda i: (0, i, 0)),
            pl.BlockSpec((BLK, D), lambda i: (i, 0)),
        ],
        out_specs=[
            pl.BlockSpec((BLK, 1), lambda i: (i, 0)),
            pl.BlockSpec((BLK, D), lambda i: (i, 0)),
            pl.BlockSpec((BLK, D), lambda i: (i, 0)),
        ],
        out_shape=[
            jax.ShapeDtypeStruct((N, 1), jnp.float32),
            jax.ShapeDtypeStruct((N, D), jnp.float32),
            jax.ShapeDtypeStruct((N, D), jnp.float32),
        ],
    )(pdeg, init)


def _blend_body(emit_g, p_ref, hpre_ref, r_ref, norm_ref, *out_refs):
    norm = norm_ref[...]
    h = ALPHA * ((p_ref[0] + p_ref[1]) * norm) + r_ref[...] + (1.0 - ALPHA) * hpre_ref[...]
    out_refs[0][...] = h
    if emit_g:
        out_refs[1][...] = h * norm


def _blend(p, hpre, r, norm, emit_g):
    n_out = 2 if emit_g else 1
    return pl.pallas_call(
        functools.partial(_blend_body, emit_g),
        grid=(N // BLK,),
        in_specs=[
            pl.BlockSpec((NC, BLK, D), lambda i: (0, i, 0)),
            pl.BlockSpec((BLK, D), lambda i: (i, 0)),
            pl.BlockSpec((BLK, D), lambda i: (i, 0)),
            pl.BlockSpec((BLK, 1), lambda i: (i, 0)),
        ],
        out_specs=[pl.BlockSpec((BLK, D), lambda i: (i, 0))] * n_out,
        out_shape=[jax.ShapeDtypeStruct((N, D), jnp.float32)] * n_out,
    )(p, hpre, r, norm)


def kernel(features, initial_features, edge_index):
    del features
    ei = edge_index.astype(jnp.int32)
    npad_e = EPAD - E
    pad_i = jnp.arange(npad_e, dtype=jnp.int32)
    src = jnp.concatenate([ei[0], (pad_i * 97) % N]).reshape(NROWS, CH)
    dst = jnp.concatenate([ei[1], N + pad_i % (NPAD - N)]).reshape(NROWS, CH)

    pdeg = _deg(dst)
    norm, g, r = _prep(pdeg, initial_features)

    h_pre = initial_features
    for step in range(K):
        p = _prop(g, src, dst)
        out = _blend(p, h_pre, r, norm, emit_g=(step < K - 1))
        if step < K - 1:
            h_pre, g = out
        else:
            (h,) = out
    return h
